# baseline (speedup 1.0000x reference)
import functools

import jax
import jax.numpy as jnp
from jax import lax
from jax.experimental import pallas as pl
from jax.experimental.pallas import tpu as pltpu
from jax.experimental.pallas import tpu_sc as plsc

N = 10000
E = 320000
D = 128
NUM_LABELS = 40

NC = 2
NS = 16
NW = NC * NS
CHUNK = 80
NPAIR = 64
NCHUNK = 2 * NPAIR
EPW = NCHUNK * CHUNK
PAD_W = EPW - E // NW
N_JUNK = PAD_W
ROWS_A = 624
TAIL = N - NS * ROWS_A


def _agg_body(h_hbm, ei_hbm, zeros_hbm, out_hbm,
              acc_sh, idx_a, idx_b, rows_a, rows_b,
              semz, semia, semib, sema, semb):
    c = lax.axis_index("c")
    s = lax.axis_index("s")
    wid = s * NC + c

    zcp = pltpu.make_async_copy(zeros_hbm, acc_sh.at[pl.ds(s * ROWS_A, ROWS_A)],
                                semz)
    zcp.start()
    zcp.wait()

    @pl.when(s == NS - 1)
    def _():
        pltpu.sync_copy(zeros_hbm.at[pl.ds(0, TAIL)],
                        acc_sh.at[pl.ds(NS * ROWS_A, TAIL)])

    plsc.subcore_barrier()

    def load_pair(j, ibuf, sem):
        pltpu.make_async_copy(ei_hbm.at[wid, j], ibuf, sem).start()

    def wait_pair(ibuf, sem):
        pltpu.make_async_copy(ei_hbm.at[0, 0], ibuf, sem).wait()

    def start_gather(sv, buf, sem):
        pltpu.make_async_copy(h_hbm.at[sv], buf, sem).start()

    def wait_gather(buf, sem):
        pltpu.make_async_copy(h_hbm.at[pl.ds(0, CHUNK)], buf, sem).wait()

    def scatter_add(dv, buf):
        pltpu.sync_copy(buf, acc_sh.at[dv], add=True)

    load_pair(0, idx_a, semia)
    wait_pair(idx_a, semia)
    start_gather(idx_a.at[0, 0], rows_a, sema)
    load_pair(1, idx_b, semib)

    @pl.loop(0, NPAIR - 2, step=2)
    def _(j):
        wait_gather(rows_a, sema)
        start_gather(idx_a.at[1, 0], rows_b, semb)
        scatter_add(idx_a.at[0, 1], rows_a)
        wait_gather(rows_b, semb)
        wait_pair(idx_b, semib)
        start_gather(idx_b.at[0, 0], rows_a, sema)
        scatter_add(idx_a.at[1, 1], rows_b)
        load_pair(j + 2, idx_a, semia)
        wait_gather(rows_a, sema)
        start_gather(idx_b.at[1, 0], rows_b, semb)
        scatter_add(idx_b.at[0, 1], rows_a)
        wait_gather(rows_b, semb)
        wait_pair(idx_a, semia)
        start_gather(idx_a.at[0, 0], rows_a, sema)
        scatter_add(idx_b.at[1, 1], rows_b)
        load_pair(j + 3, idx_b, semib)

    wait_gather(rows_a, sema)
    start_gather(idx_a.at[1, 0], rows_b, semb)
    scatter_add(idx_a.at[0, 1], rows_a)
    wait_gather(rows_b, semb)
    wait_pair(idx_b, semib)
    start_gather(idx_b.at[0, 0], rows_a, sema)
    scatter_add(idx_a.at[1, 1], rows_b)
    wait_gather(rows_a, sema)
    start_gather(idx_b.at[1, 0], rows_b, semb)
    scatter_add(idx_b.at[0, 1], rows_a)
    wait_gather(rows_b, semb)
    scatter_add(idx_b.at[1, 1], rows_b)

    plsc.subcore_barrier()
    pltpu.sync_copy(acc_sh.at[pl.ds(s * ROWS_A, ROWS_A)],
                    out_hbm.at[c, pl.ds(s * ROWS_A, ROWS_A)])

    @pl.when(s == NS - 1)
    def _():
        pltpu.sync_copy(acc_sh.at[pl.ds(NS * ROWS_A, TAIL)],
                        out_hbm.at[c, pl.ds(NS * ROWS_A, TAIL)])


def _sc_aggregate(h, ei, zeros):
    mesh = plsc.VectorSubcoreMesh(core_axis_name="c", subcore_axis_name="s")
    kfn = pl.kernel(
        _agg_body,
        out_type=jax.ShapeDtypeStruct((NC, N, D), jnp.float32),
        mesh=mesh,
        scratch_types=[
            pltpu.VMEM_SHARED((N + N_JUNK, D), jnp.float32),
            pltpu.VMEM((2, 2, CHUNK), jnp.int32),
            pltpu.VMEM((2, 2, CHUNK), jnp.int32),
            pltpu.VMEM((CHUNK, D), jnp.float32),
            pltpu.VMEM((CHUNK, D), jnp.float32),
            pltpu.SemaphoreType.DMA,
            pltpu.SemaphoreType.DMA,
            pltpu.SemaphoreType.DMA,
            pltpu.SemaphoreType.DMA,
            pltpu.SemaphoreType.DMA,
        ],
    )
    return kfn(h, ei, zeros)


BR = 1000


def _mlp_block(parts_ref, h_ref, W1_ref, b1_ref, W2_ref, b2_ref, o_ref):
    z = h_ref[...] + parts_ref[0] + parts_ref[1]
    z = jnp.dot(z, W1_ref[...], preferred_element_type=jnp.float32) + b1_ref[...]
    z = jnp.maximum(z, 0.0)
    o_ref[...] = jnp.dot(z, W2_ref[...], preferred_element_type=jnp.float32) + b2_ref[...]


def _mlp_final_block(parts_ref, h_ref, W1_ref, b1_ref, W2_ref, b2_ref,
                     Wc_ref, bc_ref, o_ref, logits_ref):
    z = h_ref[...] + parts_ref[0] + parts_ref[1]
    z = jnp.dot(z, W1_ref[...], preferred_element_type=jnp.float32) + b1_ref[...]
    z = jnp.maximum(z, 0.0)
    h_out = jnp.dot(z, W2_ref[...], preferred_element_type=jnp.float32) + b2_ref[...]
    o_ref[...] = h_out
    logits_ref[...] = (jnp.dot(h_out, Wc_ref[...], preferred_element_type=jnp.float32)
                       + bc_ref[...])


def _row_spec(block_rows, cols):
    return pl.BlockSpec((block_rows, cols), lambda i: (i, 0))


def _full_spec(shape):
    return pl.BlockSpec(shape, lambda i: tuple(0 for _ in shape))


def _tc_mlp(parts, h, W1, b1, W2, b2):
    return pl.pallas_call(
        _mlp_block,
        grid=(N // BR,),
        in_specs=[
            pl.BlockSpec((NC, BR, D), lambda i: (0, i, 0)),
            _row_spec(BR, D),
            _full_spec((D, D)),
            _full_spec((1, D)),
            _full_spec((D, D)),
            _full_spec((1, D)),
        ],
        out_specs=_row_spec(BR, D),
        out_shape=jax.ShapeDtypeStruct((N, D), jnp.float32),
    )(parts, h, W1, b1.reshape(1, D), W2, b2.reshape(1, D))


def _tc_mlp_final(parts, h, W1, b1, W2, b2, Wc, bc):
    return pl.pallas_call(
        _mlp_final_block,
        grid=(N // BR,),
        in_specs=[
            pl.BlockSpec((NC, BR, D), lambda i: (0, i, 0)),
            _row_spec(BR, D),
            _full_spec((D, D)),
            _full_spec((1, D)),
            _full_spec((D, D)),
            _full_spec((1, D)),
            _full_spec((D, NUM_LABELS)),
            _full_spec((1, NUM_LABELS)),
        ],
        out_specs=[_row_spec(BR, D), _row_spec(BR, NUM_LABELS)],
        out_shape=[jax.ShapeDtypeStruct((N, D), jnp.float32),
                   jax.ShapeDtypeStruct((N, NUM_LABELS), jnp.float32)],
    )(parts, h, W1, b1.reshape(1, D), W2, b2.reshape(1, D),
      Wc, bc.reshape(1, NUM_LABELS))


def kernel(x, edge_index, W1_0, b1_0, W2_0, b2_0, W1_1, b1_1, W2_1, b2_1,
           W1_2, b1_2, W2_2, b2_2, Wc, bc):
    epw_real = E // NW
    srcw = jnp.concatenate(
        [edge_index[0].reshape(NW, epw_real),
         jnp.zeros((NW, PAD_W), jnp.int32)], axis=1)
    dstw = jnp.concatenate(
        [edge_index[1].reshape(NW, epw_real),
         jnp.broadcast_to(N + jnp.arange(PAD_W, dtype=jnp.int32),
                          (NW, PAD_W))], axis=1)
    ei = jnp.stack([srcw.reshape(NW, NPAIR, 2, CHUNK),
                    dstw.reshape(NW, NPAIR, 2, CHUNK)], axis=3)
    zeros = jnp.zeros((ROWS_A, D), jnp.float32)

    h = x
    parts = _sc_aggregate(h, ei, zeros)
    h = _tc_mlp(parts, h, W1_0, b1_0, W2_0, b2_0)
    parts = _sc_aggregate(h, ei, zeros)
    h = _tc_mlp(parts, h, W1_1, b1_1, W2_1, b2_1)
    parts = _sc_aggregate(h, ei, zeros)
    h, logits = _tc_mlp_final(parts, h, W1_2, b1_2, W2_2, b2_2, Wc, bc)
    return (h, logits)


# speedup vs baseline: 3.2589x; 3.2589x over previous
import functools

import jax
import jax.numpy as jnp
from jax import lax
from jax.experimental import pallas as pl
from jax.experimental.pallas import tpu as pltpu
from jax.experimental.pallas import tpu_sc as plsc

N = 10000
E = 320000
D = 128
NUM_LABELS = 40

NC = 2
NS = 16
NW = NC * NS
CHUNK = 80
EPW = E // NW
NCHUNK = EPW // CHUNK
ROWS_A = 624
TAIL = N - NS * ROWS_A


def _agg_body(h_hbm, src_hbm, dst_hbm, zeros_hbm, out_hbm,
              acc_sh,
              src_a0, src_a1, dst_a0, dst_a1,
              src_b0, src_b1, dst_b0, dst_b1,
              rows_a, rows_b, rows_c, rows_d,
              semz, semia, semib, sema, semb, semc, semd):
    c = lax.axis_index("c")
    s = lax.axis_index("s")
    wid = s * NC + c

    zcp = pltpu.make_async_copy(zeros_hbm, acc_sh.at[pl.ds(s * ROWS_A, ROWS_A)],
                                semz)
    zcp.start()
    zcp.wait()

    @pl.when(s == NS - 1)
    def _():
        pltpu.sync_copy(zeros_hbm.at[pl.ds(0, TAIL)],
                        acc_sh.at[pl.ds(NS * ROWS_A, TAIL)])

    plsc.subcore_barrier()

    base_w = wid * EPW

    def start_idx(i, sv, dv, sem):
        pltpu.make_async_copy(src_hbm.at[pl.ds(base_w + i * CHUNK, CHUNK)],
                              sv, sem).start()
        pltpu.make_async_copy(dst_hbm.at[pl.ds(base_w + i * CHUNK, CHUNK)],
                              dv, sem).start()

    def wait_idx(sv, dv, sem):
        pltpu.make_async_copy(src_hbm.at[pl.ds(0, CHUNK)], sv, sem).wait()
        pltpu.make_async_copy(src_hbm.at[pl.ds(0, CHUNK)], dv, sem).wait()

    def start_gather(sv, buf, sem):
        pltpu.make_async_copy(h_hbm.at[sv], buf, sem).start()

    def wait_gather(buf, sem):
        pltpu.make_async_copy(h_hbm.at[pl.ds(0, CHUNK)], buf, sem).wait()

    def scatter_add(dv, buf):
        pltpu.sync_copy(buf, acc_sh.at[dv], add=True)

    start_idx(NCHUNK - 1, src_a0, dst_a0, semia)
    wait_idx(src_a0, dst_a0, semia)
    start_gather(src_a0, rows_a, sema)
    wait_gather(rows_a, sema)
    scatter_add(dst_a0, rows_a)

    start_idx(0, src_a0, dst_a0, semia)
    start_idx(1, src_a1, dst_a1, semia)
    start_idx(2, src_b0, dst_b0, semib)
    start_idx(3, src_b1, dst_b1, semib)
    wait_idx(src_a0, dst_a0, semia)
    wait_idx(src_a1, dst_a1, semia)
    start_gather(src_a0, rows_a, sema)
    start_gather(src_a1, rows_b, semb)
    wait_idx(src_b0, dst_b0, semib)
    wait_idx(src_b1, dst_b1, semib)

    @pl.loop(0, NCHUNK - 9, step=4)
    def _(i):
        start_gather(src_b0, rows_c, semc)
        start_gather(src_b1, rows_d, semd)
        wait_gather(rows_a, sema)
        scatter_add(dst_a0, rows_a)
        wait_gather(rows_b, semb)
        scatter_add(dst_a1, rows_b)
        start_idx(i + 4, src_a0, dst_a0, semia)
        start_idx(i + 5, src_a1, dst_a1, semia)
        wait_idx(src_a0, dst_a0, semia)
        wait_idx(src_a1, dst_a1, semia)
        start_gather(src_a0, rows_a, sema)
        start_gather(src_a1, rows_b, semb)
        wait_gather(rows_c, semc)
        scatter_add(dst_b0, rows_c)
        wait_gather(rows_d, semd)
        scatter_add(dst_b1, rows_d)
        start_idx(i + 6, src_b0, dst_b0, semib)
        start_idx(i + 7, src_b1, dst_b1, semib)
        wait_idx(src_b0, dst_b0, semib)
        wait_idx(src_b1, dst_b1, semib)

    start_gather(src_b0, rows_c, semc)
    start_gather(src_b1, rows_d, semd)
    wait_gather(rows_a, sema)
    scatter_add(dst_a0, rows_a)
    wait_gather(rows_b, semb)
    scatter_add(dst_a1, rows_b)
    start_idx(NCHUNK - 5, src_a0, dst_a0, semia)
    start_idx(NCHUNK - 4, src_a1, dst_a1, semia)
    wait_idx(src_a0, dst_a0, semia)
    wait_idx(src_a1, dst_a1, semia)
    start_gather(src_a0, rows_a, sema)
    start_gather(src_a1, rows_b, semb)
    wait_gather(rows_c, semc)
    scatter_add(dst_b0, rows_c)
    wait_gather(rows_d, semd)
    scatter_add(dst_b1, rows_d)
    start_idx(NCHUNK - 3, src_b0, dst_b0, semib)
    start_idx(NCHUNK - 2, src_b1, dst_b1, semib)
    wait_idx(src_b0, dst_b0, semib)
    wait_idx(src_b1, dst_b1, semib)
    start_gather(src_b0, rows_c, semc)
    start_gather(src_b1, rows_d, semd)
    wait_gather(rows_a, sema)
    scatter_add(dst_a0, rows_a)
    wait_gather(rows_b, semb)
    scatter_add(dst_a1, rows_b)
    wait_gather(rows_c, semc)
    scatter_add(dst_b0, rows_c)
    wait_gather(rows_d, semd)
    scatter_add(dst_b1, rows_d)

    plsc.subcore_barrier()
    pltpu.sync_copy(acc_sh.at[pl.ds(s * ROWS_A, ROWS_A)],
                    out_hbm.at[c, pl.ds(s * ROWS_A, ROWS_A)])

    @pl.when(s == NS - 1)
    def _():
        pltpu.sync_copy(acc_sh.at[pl.ds(NS * ROWS_A, TAIL)],
                        out_hbm.at[c, pl.ds(NS * ROWS_A, TAIL)])


def _sc_aggregate(h, src, dst, zeros):
    mesh = plsc.VectorSubcoreMesh(core_axis_name="c", subcore_axis_name="s")
    kfn = pl.kernel(
        _agg_body,
        out_type=jax.ShapeDtypeStruct((NC, N, D), jnp.float32),
        mesh=mesh,
        scratch_types=(
            [pltpu.VMEM_SHARED((N, D), jnp.float32)]
            + [pltpu.VMEM((CHUNK,), jnp.int32)] * 8
            + [pltpu.VMEM((CHUNK, D), jnp.float32)] * 4
            + [pltpu.SemaphoreType.DMA] * 7
        ),
    )
    return kfn(h, src, dst, zeros)


BR = 1000


def _mlp_block(parts_ref, h_ref, W1_ref, b1_ref, W2_ref, b2_ref, o_ref):
    z = h_ref[...] + parts_ref[0] + parts_ref[1]
    z = jnp.dot(z, W1_ref[...], preferred_element_type=jnp.float32) + b1_ref[...]
    z = jnp.maximum(z, 0.0)
    o_ref[...] = jnp.dot(z, W2_ref[...], preferred_element_type=jnp.float32) + b2_ref[...]


def _mlp_final_block(parts_ref, h_ref, W1_ref, b1_ref, W2_ref, b2_ref,
                     Wc_ref, bc_ref, o_ref, logits_ref):
    z = h_ref[...] + parts_ref[0] + parts_ref[1]
    z = jnp.dot(z, W1_ref[...], preferred_element_type=jnp.float32) + b1_ref[...]
    z = jnp.maximum(z, 0.0)
    h_out = jnp.dot(z, W2_ref[...], preferred_element_type=jnp.float32) + b2_ref[...]
    o_ref[...] = h_out
    logits_ref[...] = (jnp.dot(h_out, Wc_ref[...], preferred_element_type=jnp.float32)
                       + bc_ref[...])


def _row_spec(block_rows, cols):
    return pl.BlockSpec((block_rows, cols), lambda i: (i, 0))


def _full_spec(shape):
    return pl.BlockSpec(shape, lambda i: tuple(0 for _ in shape))


def _tc_mlp(parts, h, W1, b1, W2, b2):
    return pl.pallas_call(
        _mlp_block,
        grid=(N // BR,),
        in_specs=[
            pl.BlockSpec((NC, BR, D), lambda i: (0, i, 0)),
            _row_spec(BR, D),
            _full_spec((D, D)),
            _full_spec((1, D)),
            _full_spec((D, D)),
            _full_spec((1, D)),
        ],
        out_specs=_row_spec(BR, D),
        out_shape=jax.ShapeDtypeStruct((N, D), jnp.float32),
    )(parts, h, W1, b1.reshape(1, D), W2, b2.reshape(1, D))


def _tc_mlp_final(parts, h, W1, b1, W2, b2, Wc, bc):
    return pl.pallas_call(
        _mlp_final_block,
        grid=(N // BR,),
        in_specs=[
            pl.BlockSpec((NC, BR, D), lambda i: (0, i, 0)),
            _row_spec(BR, D),
            _full_spec((D, D)),
            _full_spec((1, D)),
            _full_spec((D, D)),
            _full_spec((1, D)),
            _full_spec((D, NUM_LABELS)),
            _full_spec((1, NUM_LABELS)),
        ],
        out_specs=[_row_spec(BR, D), _row_spec(BR, NUM_LABELS)],
        out_shape=[jax.ShapeDtypeStruct((N, D), jnp.float32),
                   jax.ShapeDtypeStruct((N, NUM_LABELS), jnp.float32)],
    )(parts, h, W1, b1.reshape(1, D), W2, b2.reshape(1, D),
      Wc, bc.reshape(1, NUM_LABELS))


def kernel(x, edge_index, W1_0, b1_0, W2_0, b2_0, W1_1, b1_1, W2_1, b2_1,
           W1_2, b1_2, W2_2, b2_2, Wc, bc):
    src = edge_index[0]
    dst = edge_index[1]
    zeros = jnp.zeros((ROWS_A, D), jnp.float32)

    h = x
    parts = _sc_aggregate(h, src, dst, zeros)
    h = _tc_mlp(parts, h, W1_0, b1_0, W2_0, b2_0)
    parts = _sc_aggregate(h, src, dst, zeros)
    h = _tc_mlp(parts, h, W1_1, b1_1, W2_1, b2_1)
    parts = _sc_aggregate(h, src, dst, zeros)
    h, logits = _tc_mlp_final(parts, h, W1_2, b1_2, W2_2, b2_2, Wc, bc)
    return (h, logits)


# speedup vs baseline: 3.4856x; 1.0696x over previous
import functools

import jax
import jax.numpy as jnp
from jax import lax
from jax.experimental import pallas as pl
from jax.experimental.pallas import tpu as pltpu
from jax.experimental.pallas import tpu_sc as plsc

N = 10000
E = 320000
D = 128
NUM_LABELS = 40

NC = 2
NS = 16
NW = NC * NS
CHUNK = 80
EPW = E // NW
NCHUNK = EPW // CHUNK
ROWS_A = 624
TAIL = N - NS * ROWS_A


def _agg_body(h_hbm, src_hbm, dst_hbm, zeros_hbm, out_hbm,
              acc_sh,
              src_a0, src_a1, dst_a0, dst_a1,
              src_b0, src_b1, dst_b0, dst_b1,
              src_c0, src_c1, dst_c0, dst_c1,
              src_d0, src_d1, dst_d0, dst_d1,
              rows_a, rows_b, rows_c, rows_d,
              semz, semia, semib, semic, semid, sema, semb, semc, semd):
    c = lax.axis_index("c")
    s = lax.axis_index("s")
    wid = s * NC + c

    zcp = pltpu.make_async_copy(zeros_hbm, acc_sh.at[pl.ds(s * ROWS_A, ROWS_A)],
                                semz)
    zcp.start()
    zcp.wait()

    @pl.when(s == NS - 1)
    def _():
        pltpu.sync_copy(zeros_hbm.at[pl.ds(0, TAIL)],
                        acc_sh.at[pl.ds(NS * ROWS_A, TAIL)])

    plsc.subcore_barrier()

    base_w = wid * EPW

    def start_idx(i, sv, dv, sem):
        pltpu.make_async_copy(src_hbm.at[pl.ds(base_w + i * CHUNK, CHUNK)],
                              sv, sem).start()
        pltpu.make_async_copy(dst_hbm.at[pl.ds(base_w + i * CHUNK, CHUNK)],
                              dv, sem).start()

    def wait_idx(sv, dv, sem):
        pltpu.make_async_copy(src_hbm.at[pl.ds(0, CHUNK)], sv, sem).wait()
        pltpu.make_async_copy(src_hbm.at[pl.ds(0, CHUNK)], dv, sem).wait()

    def start_gather(sv, buf, sem):
        pltpu.make_async_copy(h_hbm.at[sv], buf, sem).start()

    def wait_gather(buf, sem):
        pltpu.make_async_copy(h_hbm.at[pl.ds(0, CHUNK)], buf, sem).wait()

    def scatter_add(dv, buf):
        pltpu.sync_copy(buf, acc_sh.at[dv], add=True)

    start_idx(NCHUNK - 1, src_a0, dst_a0, semia)
    wait_idx(src_a0, dst_a0, semia)
    start_gather(src_a0, rows_a, sema)
    wait_gather(rows_a, sema)
    scatter_add(dst_a0, rows_a)

    start_idx(0, src_a0, dst_a0, semia)
    start_idx(1, src_a1, dst_a1, semia)
    start_idx(2, src_b0, dst_b0, semib)
    start_idx(3, src_b1, dst_b1, semib)
    start_idx(4, src_c0, dst_c0, semic)
    start_idx(5, src_c1, dst_c1, semic)
    start_idx(6, src_d0, dst_d0, semid)
    start_idx(7, src_d1, dst_d1, semid)
    wait_idx(src_a0, dst_a0, semia)
    wait_idx(src_a1, dst_a1, semia)
    start_gather(src_a0, rows_a, sema)
    start_gather(src_a1, rows_b, semb)
    wait_idx(src_b0, dst_b0, semib)
    wait_idx(src_b1, dst_b1, semib)

    @pl.loop(0, NCHUNK - 5, step=8)
    def _(i):
        start_gather(src_b0, rows_c, semc)
        start_gather(src_b1, rows_d, semd)
        wait_gather(rows_a, sema)
        scatter_add(dst_a0, rows_a)
        wait_gather(rows_b, semb)
        scatter_add(dst_a1, rows_b)
        start_idx(i + 8, src_a0, dst_a0, semia)
        start_idx(i + 9, src_a1, dst_a1, semia)
        wait_idx(src_c0, dst_c0, semic)
        wait_idx(src_c1, dst_c1, semic)
        start_gather(src_c0, rows_a, sema)
        start_gather(src_c1, rows_b, semb)
        wait_gather(rows_c, semc)
        scatter_add(dst_b0, rows_c)
        wait_gather(rows_d, semd)
        scatter_add(dst_b1, rows_d)
        start_idx(i + 10, src_b0, dst_b0, semib)
        start_idx(i + 11, src_b1, dst_b1, semib)
        wait_idx(src_d0, dst_d0, semid)
        wait_idx(src_d1, dst_d1, semid)
        start_gather(src_d0, rows_c, semc)
        start_gather(src_d1, rows_d, semd)
        wait_gather(rows_a, sema)
        scatter_add(dst_c0, rows_a)
        wait_gather(rows_b, semb)
        scatter_add(dst_c1, rows_b)

        @pl.when(i < NCHUNK - 14)
        def _():
            start_idx(i + 12, src_c0, dst_c0, semic)
            start_idx(i + 13, src_c1, dst_c1, semic)

        wait_idx(src_a0, dst_a0, semia)
        wait_idx(src_a1, dst_a1, semia)
        start_gather(src_a0, rows_a, sema)
        start_gather(src_a1, rows_b, semb)
        wait_gather(rows_c, semc)
        scatter_add(dst_d0, rows_c)
        wait_gather(rows_d, semd)
        scatter_add(dst_d1, rows_d)

        @pl.when(i < NCHUNK - 16)
        def _():
            start_idx(i + 14, src_d0, dst_d0, semid)
            start_idx(i + 15, src_d1, dst_d1, semid)

        wait_idx(src_b0, dst_b0, semib)
        wait_idx(src_b1, dst_b1, semib)

    start_gather(src_b0, rows_c, semc)
    start_gather(src_b1, rows_d, semd)
    wait_gather(rows_a, sema)
    scatter_add(dst_a0, rows_a)
    wait_gather(rows_b, semb)
    scatter_add(dst_a1, rows_b)
    wait_gather(rows_c, semc)
    scatter_add(dst_b0, rows_c)
    wait_gather(rows_d, semd)
    scatter_add(dst_b1, rows_d)

    plsc.subcore_barrier()
    pltpu.sync_copy(acc_sh.at[pl.ds(s * ROWS_A, ROWS_A)],
                    out_hbm.at[c, pl.ds(s * ROWS_A, ROWS_A)])

    @pl.when(s == NS - 1)
    def _():
        pltpu.sync_copy(acc_sh.at[pl.ds(NS * ROWS_A, TAIL)],
                        out_hbm.at[c, pl.ds(NS * ROWS_A, TAIL)])


def _sc_aggregate(h, src, dst, zeros):
    mesh = plsc.VectorSubcoreMesh(core_axis_name="c", subcore_axis_name="s")
    kfn = pl.kernel(
        _agg_body,
        out_type=jax.ShapeDtypeStruct((NC, N, D), jnp.float32),
        mesh=mesh,
        scratch_types=(
            [pltpu.VMEM_SHARED((N, D), jnp.float32)]
            + [pltpu.VMEM((CHUNK,), jnp.int32)] * 16
            + [pltpu.VMEM((CHUNK, D), jnp.float32)] * 4
            + [pltpu.SemaphoreType.DMA] * 9
        ),
    )
    return kfn(h, src, dst, zeros)


BR = 1000


def _mlp_block(parts_ref, h_ref, W1_ref, b1_ref, W2_ref, b2_ref, o_ref):
    z = h_ref[...] + parts_ref[0] + parts_ref[1]
    z = jnp.dot(z, W1_ref[...], preferred_element_type=jnp.float32) + b1_ref[...]
    z = jnp.maximum(z, 0.0)
    o_ref[...] = jnp.dot(z, W2_ref[...], preferred_element_type=jnp.float32) + b2_ref[...]


def _mlp_final_block(parts_ref, h_ref, W1_ref, b1_ref, W2_ref, b2_ref,
                     Wc_ref, bc_ref, o_ref, logits_ref):
    z = h_ref[...] + parts_ref[0] + parts_ref[1]
    z = jnp.dot(z, W1_ref[...], preferred_element_type=jnp.float32) + b1_ref[...]
    z = jnp.maximum(z, 0.0)
    h_out = jnp.dot(z, W2_ref[...], preferred_element_type=jnp.float32) + b2_ref[...]
    o_ref[...] = h_out
    logits_ref[...] = (jnp.dot(h_out, Wc_ref[...], preferred_element_type=jnp.float32)
                       + bc_ref[...])


def _row_spec(block_rows, cols):
    return pl.BlockSpec((block_rows, cols), lambda i: (i, 0))


def _full_spec(shape):
    return pl.BlockSpec(shape, lambda i: tuple(0 for _ in shape))


def _tc_mlp(parts, h, W1, b1, W2, b2):
    return pl.pallas_call(
        _mlp_block,
        grid=(N // BR,),
        in_specs=[
            pl.BlockSpec((NC, BR, D), lambda i: (0, i, 0)),
            _row_spec(BR, D),
            _full_spec((D, D)),
            _full_spec((1, D)),
            _full_spec((D, D)),
            _full_spec((1, D)),
        ],
        out_specs=_row_spec(BR, D),
        out_shape=jax.ShapeDtypeStruct((N, D), jnp.float32),
    )(parts, h, W1, b1.reshape(1, D), W2, b2.reshape(1, D))


def _tc_mlp_final(parts, h, W1, b1, W2, b2, Wc, bc):
    return pl.pallas_call(
        _mlp_final_block,
        grid=(N // BR,),
        in_specs=[
            pl.BlockSpec((NC, BR, D), lambda i: (0, i, 0)),
            _row_spec(BR, D),
            _full_spec((D, D)),
            _full_spec((1, D)),
            _full_spec((D, D)),
            _full_spec((1, D)),
            _full_spec((D, NUM_LABELS)),
            _full_spec((1, NUM_LABELS)),
        ],
        out_specs=[_row_spec(BR, D), _row_spec(BR, NUM_LABELS)],
        out_shape=[jax.ShapeDtypeStruct((N, D), jnp.float32),
                   jax.ShapeDtypeStruct((N, NUM_LABELS), jnp.float32)],
    )(parts, h, W1, b1.reshape(1, D), W2, b2.reshape(1, D),
      Wc, bc.reshape(1, NUM_LABELS))


def kernel(x, edge_index, W1_0, b1_0, W2_0, b2_0, W1_1, b1_1, W2_1, b2_1,
           W1_2, b1_2, W2_2, b2_2, Wc, bc):
    src = edge_index[0]
    dst = edge_index[1]
    zeros = jnp.zeros((ROWS_A, D), jnp.float32)

    h = x
    parts = _sc_aggregate(h, src, dst, zeros)
    h = _tc_mlp(parts, h, W1_0, b1_0, W2_0, b2_0)
    parts = _sc_aggregate(h, src, dst, zeros)
    h = _tc_mlp(parts, h, W1_1, b1_1, W2_1, b2_1)
    parts = _sc_aggregate(h, src, dst, zeros)
    h, logits = _tc_mlp_final(parts, h, W1_2, b1_2, W2_2, b2_2, Wc, bc)
    return (h, logits)
